# baseline (speedup 1.0000x reference)
import functools

import jax
import jax.numpy as jnp
from jax import lax
from jax.experimental import pallas as pl
from jax.experimental.pallas import tpu as pltpu

N = 2560
D = 256
ROW_BLK = 1280
NUM_BLK = N // ROW_BLK
GROUPS = N // 5
TOPK = 110
LANES = 128
SUBL = N // LANES


def _body(in_ref, ptl_ref, gmask_ref, w_in_ref, b_in_ref, w_q_ref, b_q_ref,
          w_k_ref, b_k_ref, out_ref, q_s, k_s, p_s, dist_s):
    t = pl.program_id(0)

    @pl.when(t < 4)
    def flash_phase():
        pts = in_ref[t]
        x = jnp.dot(pts, w_in_ref[...], preferred_element_type=jnp.float32)
        x = x + b_in_ref[...]
        q = jnp.dot(x, w_q_ref[...], preferred_element_type=jnp.float32) + b_q_ref[...]
        k = jnp.dot(x, w_k_ref[...], preferred_element_type=jnp.float32) + b_k_ref[...]
        q_s[...] = q / 16.0
        k_s[...] = k

        ident = (lax.broadcasted_iota(jnp.int32, (ROW_BLK, ROW_BLK), 0) ==
                 lax.broadcasted_iota(jnp.int32, (ROW_BLK, ROW_BLK), 1))

        for i in range(NUM_BLK):
            qb = q_s[i * ROW_BLK:(i + 1) * ROW_BLK, :]
            s = lax.dot_general(qb, k_s[...], (((1,), (1,)), ((), ())),
                                preferred_element_type=jnp.float32)
            e = jnp.exp(s)
            l = jnp.sum(e, axis=1, keepdims=True)
            sblk = s[:, i * ROW_BLK:(i + 1) * ROW_BLK]
            diag_s = jnp.sum(jnp.where(ident, sblk, 0.0),
                             axis=1, keepdims=True)
            p_s[i * ROW_BLK:(i + 1) * ROW_BLK, :] = jnp.exp(diag_s) / l

        gs = jnp.dot(gmask_ref[...], p_s[...],
                     preferred_element_type=jnp.float32,
                     precision=lax.Precision.HIGHEST)
        gbest = jnp.max(gs)
        gidx = lax.broadcasted_iota(jnp.int32, (GROUPS, 1), 0)
        g = jnp.min(jnp.where(gs == gbest, gidx, GROUPS))

        group_pts = in_ref[t, pl.ds(g * 5, 5), :]
        cx = jnp.sum(group_pts[:, 0:1]) / 5.0
        cy = jnp.sum(group_pts[:, 1:2]) / 5.0
        cz = jnp.sum(group_pts[:, 2:3]) / 5.0

        dx = ptl_ref[t, 0] - cx
        dy = ptl_ref[t, 1] - cy
        dz = ptl_ref[t, 2] - cz
        dist_s[t] = jnp.sqrt(dx * dx + dy * dy + dz * dz)

    @pl.when(t == 4)
    def select_phase():
        lin = (lax.broadcasted_iota(jnp.int32, (SUBL, LANES), 0) * LANES +
               lax.broadcasted_iota(jnp.int32, (SUBL, LANES), 1))

        def pick(ti, carry):
            ds_ = list(carry)
            for b in range(4):
                db = ds_[b]
                mn = jnp.min(db)
                ib = jnp.min(jnp.where(db == mn, lin, N))
                out_ref[b, pl.ds(ti, 1), :] = in_ref[b, pl.ds(ib, 1), :]
                ds_[b] = jnp.where(lin == ib, jnp.inf, db)
            return tuple(ds_)

        lax.fori_loop(0, TOPK, pick,
                      tuple(dist_s[b] for b in range(4)))


@functools.partial(jax.jit, static_argnames=("interpret",))
def _run(in_mat, W_in, b_in, W_q, b_q, W_k, b_k, interpret=False):
    B = in_mat.shape[0]
    pts_lanes = in_mat.transpose(0, 2, 1).reshape(B, 3, SUBL, LANES)
    gcol = jnp.arange(N, dtype=jnp.int32)[None, :]
    grow = jnp.arange(GROUPS, dtype=jnp.int32)[:, None]
    gmask = jnp.where((gcol >= grow * 5) & (gcol < grow * 5 + 5), 1.0, 0.0)
    in_specs = [
        pl.BlockSpec((B, N, 3), lambda t: (0, 0, 0)),
        pl.BlockSpec((B, 3, SUBL, LANES), lambda t: (0, 0, 0, 0)),
        pl.BlockSpec((GROUPS, N), lambda t: (0, 0)),
        pl.BlockSpec((3, D), lambda t: (0, 0)),
        pl.BlockSpec((1, D), lambda t: (0, 0)),
        pl.BlockSpec((D, D), lambda t: (0, 0)),
        pl.BlockSpec((1, D), lambda t: (0, 0)),
        pl.BlockSpec((D, D), lambda t: (0, 0)),
        pl.BlockSpec((1, D), lambda t: (0, 0)),
    ]
    return pl.pallas_call(
        _body,
        grid=(5,),
        in_specs=in_specs,
        out_specs=pl.BlockSpec((B, TOPK, 3), lambda t: (0, 0, 0)),
        out_shape=jax.ShapeDtypeStruct((B, TOPK, 3), jnp.float32),
        scratch_shapes=[
            pltpu.VMEM((N, D), jnp.float32),
            pltpu.VMEM((N, D), jnp.float32),
            pltpu.VMEM((N, 1), jnp.float32),
            pltpu.VMEM((B, SUBL, LANES), jnp.float32),
        ],
        interpret=interpret,
    )(in_mat, pts_lanes, gmask, W_in, b_in.reshape(1, D), W_q,
      b_q.reshape(1, D), W_k, b_k.reshape(1, D))


def kernel(in_mat, W_in, b_in, W_q, b_q, W_k, b_k, W_v, b_v, W_o, b_o):
    del W_v, b_v, W_o, b_o
    return _run(in_mat, W_in, b_in, W_q, b_q, W_k, b_k)


# speedup vs baseline: 1.0571x; 1.0571x over previous
import functools

import jax
import jax.numpy as jnp
from jax import lax
from jax.experimental import pallas as pl
from jax.experimental.pallas import tpu as pltpu

N = 2560
D = 256
ROW_BLK = 320
NUM_BLK = N // ROW_BLK
GROUPS = N // 5
TOPK = 110
LANES = 128
SUBL = N // LANES


def _body(in_ref, ptl_ref, gmask_ref, w_in_ref, b_in_ref, w_q_ref, b_q_ref,
          w_k_ref, b_k_ref, out_ref, q_s, k_s, p_s, dist_s):
    t = pl.program_id(0)

    @pl.when(t < 4)
    def flash_phase():
        pts = in_ref[t]
        x = jnp.dot(pts, w_in_ref[...], preferred_element_type=jnp.float32)
        x = x + b_in_ref[...]
        q = jnp.dot(x, w_q_ref[...], preferred_element_type=jnp.float32) + b_q_ref[...]
        k = jnp.dot(x, w_k_ref[...], preferred_element_type=jnp.float32) + b_k_ref[...]
        q_s[...] = q / 16.0
        k_s[...] = k

        ident = (lax.broadcasted_iota(jnp.int32, (ROW_BLK, ROW_BLK), 0) ==
                 lax.broadcasted_iota(jnp.int32, (ROW_BLK, ROW_BLK), 1))

        for i in range(NUM_BLK):
            qb = q_s[i * ROW_BLK:(i + 1) * ROW_BLK, :]
            s = lax.dot_general(qb, k_s[...], (((1,), (1,)), ((), ())),
                                preferred_element_type=jnp.float32)
            e = jnp.exp(s)
            l = jnp.sum(e, axis=1, keepdims=True)
            sblk = s[:, i * ROW_BLK:(i + 1) * ROW_BLK]
            diag_s = jnp.sum(jnp.where(ident, sblk, 0.0),
                             axis=1, keepdims=True)
            p_s[i * ROW_BLK:(i + 1) * ROW_BLK, :] = jnp.exp(diag_s) / l

        gs = jnp.dot(gmask_ref[...], p_s[...],
                     preferred_element_type=jnp.float32,
                     precision=lax.Precision.HIGHEST)
        gbest = jnp.max(gs)
        gidx = lax.broadcasted_iota(jnp.int32, (GROUPS, 1), 0)
        g = jnp.min(jnp.where(gs == gbest, gidx, GROUPS))

        group_pts = in_ref[t, pl.ds(g * 5, 5), :]
        cx = jnp.sum(group_pts[:, 0:1]) / 5.0
        cy = jnp.sum(group_pts[:, 1:2]) / 5.0
        cz = jnp.sum(group_pts[:, 2:3]) / 5.0

        dx = ptl_ref[t, 0] - cx
        dy = ptl_ref[t, 1] - cy
        dz = ptl_ref[t, 2] - cz
        dist_s[t] = jnp.sqrt(dx * dx + dy * dy + dz * dz)

    @pl.when(t == 4)
    def select_phase():
        lin = (lax.broadcasted_iota(jnp.int32, (SUBL, LANES), 0) * LANES +
               lax.broadcasted_iota(jnp.int32, (SUBL, LANES), 1))

        def pick(ti, carry):
            ds_ = list(carry)
            for b in range(4):
                db = ds_[b]
                mn = jnp.min(db)
                ib = jnp.min(jnp.where(db == mn, lin, N))
                out_ref[b, pl.ds(ti, 1), :] = in_ref[b, pl.ds(ib, 1), :]
                ds_[b] = jnp.where(lin == ib, jnp.inf, db)
            return tuple(ds_)

        lax.fori_loop(0, TOPK, pick,
                      tuple(dist_s[b] for b in range(4)))


@functools.partial(jax.jit, static_argnames=("interpret",))
def _run(in_mat, W_in, b_in, W_q, b_q, W_k, b_k, interpret=False):
    B = in_mat.shape[0]
    pts_lanes = in_mat.transpose(0, 2, 1).reshape(B, 3, SUBL, LANES)
    gcol = jnp.arange(N, dtype=jnp.int32)[None, :]
    grow = jnp.arange(GROUPS, dtype=jnp.int32)[:, None]
    gmask = jnp.where((gcol >= grow * 5) & (gcol < grow * 5 + 5), 1.0, 0.0)
    in_specs = [
        pl.BlockSpec((B, N, 3), lambda t: (0, 0, 0)),
        pl.BlockSpec((B, 3, SUBL, LANES), lambda t: (0, 0, 0, 0)),
        pl.BlockSpec((GROUPS, N), lambda t: (0, 0)),
        pl.BlockSpec((3, D), lambda t: (0, 0)),
        pl.BlockSpec((1, D), lambda t: (0, 0)),
        pl.BlockSpec((D, D), lambda t: (0, 0)),
        pl.BlockSpec((1, D), lambda t: (0, 0)),
        pl.BlockSpec((D, D), lambda t: (0, 0)),
        pl.BlockSpec((1, D), lambda t: (0, 0)),
    ]
    return pl.pallas_call(
        _body,
        grid=(5,),
        in_specs=in_specs,
        out_specs=pl.BlockSpec((B, TOPK, 3), lambda t: (0, 0, 0)),
        out_shape=jax.ShapeDtypeStruct((B, TOPK, 3), jnp.float32),
        scratch_shapes=[
            pltpu.VMEM((N, D), jnp.float32),
            pltpu.VMEM((N, D), jnp.float32),
            pltpu.VMEM((N, 1), jnp.float32),
            pltpu.VMEM((B, SUBL, LANES), jnp.float32),
        ],
        interpret=interpret,
    )(in_mat, pts_lanes, gmask, W_in, b_in.reshape(1, D), W_q,
      b_q.reshape(1, D), W_k, b_k.reshape(1, D))


def kernel(in_mat, W_in, b_in, W_q, b_q, W_k, b_k, W_v, b_v, W_o, b_o):
    del W_v, b_v, W_o, b_o
    return _run(in_mat, W_in, b_in, W_q, b_q, W_k, b_k)


# speedup vs baseline: 1.0667x; 1.0091x over previous
import functools

import jax
import jax.numpy as jnp
from jax import lax
from jax.experimental import pallas as pl
from jax.experimental.pallas import tpu as pltpu

N = 2560
D = 256
ROW_BLK = 256
NUM_BLK = N // ROW_BLK
GROUPS = N // 5
TOPK = 110
LANES = 128
SUBL = N // LANES


def _body(in_ref, ptl_ref, gmask_ref, w_in_ref, b_in_ref, w_q_ref, b_q_ref,
          w_k_ref, b_k_ref, out_ref, q_s, k_s, p_s, dist_s):
    t = pl.program_id(0)

    @pl.when(t < 4)
    def flash_phase():
        pts = in_ref[t]
        x = jnp.dot(pts, w_in_ref[...], preferred_element_type=jnp.float32)
        x = x + b_in_ref[...]
        q = jnp.dot(x, w_q_ref[...], preferred_element_type=jnp.float32) + b_q_ref[...]
        k = jnp.dot(x, w_k_ref[...], preferred_element_type=jnp.float32) + b_k_ref[...]
        q_s[...] = q / 16.0
        k_s[...] = k

        ident = (lax.broadcasted_iota(jnp.int32, (ROW_BLK, ROW_BLK), 0) ==
                 lax.broadcasted_iota(jnp.int32, (ROW_BLK, ROW_BLK), 1))

        for i in range(NUM_BLK):
            qb = q_s[i * ROW_BLK:(i + 1) * ROW_BLK, :]
            s = lax.dot_general(qb, k_s[...], (((1,), (1,)), ((), ())),
                                preferred_element_type=jnp.float32)
            e = jnp.exp(s)
            l = jnp.sum(e, axis=1, keepdims=True)
            sblk = s[:, i * ROW_BLK:(i + 1) * ROW_BLK]
            diag_s = jnp.sum(jnp.where(ident, sblk, 0.0),
                             axis=1, keepdims=True)
            p_s[i * ROW_BLK:(i + 1) * ROW_BLK, :] = jnp.exp(diag_s) / l

        gs = jnp.dot(gmask_ref[...], p_s[...],
                     preferred_element_type=jnp.float32,
                     precision=lax.Precision.HIGHEST)
        gbest = jnp.max(gs)
        gidx = lax.broadcasted_iota(jnp.int32, (GROUPS, 1), 0)
        g = jnp.min(jnp.where(gs == gbest, gidx, GROUPS))

        group_pts = in_ref[t, pl.ds(g * 5, 5), :]
        cx = jnp.sum(group_pts[:, 0:1]) / 5.0
        cy = jnp.sum(group_pts[:, 1:2]) / 5.0
        cz = jnp.sum(group_pts[:, 2:3]) / 5.0

        dx = ptl_ref[t, 0] - cx
        dy = ptl_ref[t, 1] - cy
        dz = ptl_ref[t, 2] - cz
        dist_s[t] = jnp.sqrt(dx * dx + dy * dy + dz * dz)

    @pl.when(t == 4)
    def select_phase():
        lin = (lax.broadcasted_iota(jnp.int32, (SUBL, LANES), 0) * LANES +
               lax.broadcasted_iota(jnp.int32, (SUBL, LANES), 1))

        def pick(ti, carry):
            ds_ = list(carry)
            for b in range(4):
                db = ds_[b]
                mn = jnp.min(db)
                ib = jnp.min(jnp.where(db == mn, lin, N))
                out_ref[b, pl.ds(ti, 1), :] = in_ref[b, pl.ds(ib, 1), :]
                ds_[b] = jnp.where(lin == ib, jnp.inf, db)
            return tuple(ds_)

        lax.fori_loop(0, TOPK, pick,
                      tuple(dist_s[b] for b in range(4)))


@functools.partial(jax.jit, static_argnames=("interpret",))
def _run(in_mat, W_in, b_in, W_q, b_q, W_k, b_k, interpret=False):
    B = in_mat.shape[0]
    pts_lanes = in_mat.transpose(0, 2, 1).reshape(B, 3, SUBL, LANES)
    gcol = jnp.arange(N, dtype=jnp.int32)[None, :]
    grow = jnp.arange(GROUPS, dtype=jnp.int32)[:, None]
    gmask = jnp.where((gcol >= grow * 5) & (gcol < grow * 5 + 5), 1.0, 0.0)
    in_specs = [
        pl.BlockSpec((B, N, 3), lambda t: (0, 0, 0)),
        pl.BlockSpec((B, 3, SUBL, LANES), lambda t: (0, 0, 0, 0)),
        pl.BlockSpec((GROUPS, N), lambda t: (0, 0)),
        pl.BlockSpec((3, D), lambda t: (0, 0)),
        pl.BlockSpec((1, D), lambda t: (0, 0)),
        pl.BlockSpec((D, D), lambda t: (0, 0)),
        pl.BlockSpec((1, D), lambda t: (0, 0)),
        pl.BlockSpec((D, D), lambda t: (0, 0)),
        pl.BlockSpec((1, D), lambda t: (0, 0)),
    ]
    return pl.pallas_call(
        _body,
        grid=(5,),
        in_specs=in_specs,
        out_specs=pl.BlockSpec((B, TOPK, 3), lambda t: (0, 0, 0)),
        out_shape=jax.ShapeDtypeStruct((B, TOPK, 3), jnp.float32),
        scratch_shapes=[
            pltpu.VMEM((N, D), jnp.float32),
            pltpu.VMEM((N, D), jnp.float32),
            pltpu.VMEM((N, 1), jnp.float32),
            pltpu.VMEM((B, SUBL, LANES), jnp.float32),
        ],
        interpret=interpret,
    )(in_mat, pts_lanes, gmask, W_in, b_in.reshape(1, D), W_q,
      b_q.reshape(1, D), W_k, b_k.reshape(1, D))


def kernel(in_mat, W_in, b_in, W_q, b_q, W_k, b_k, W_v, b_v, W_o, b_o):
    del W_v, b_v, W_o, b_o
    return _run(in_mat, W_in, b_in, W_q, b_q, W_k, b_k)


# speedup vs baseline: 1.0722x; 1.0052x over previous
import functools

import jax
import jax.numpy as jnp
from jax import lax
from jax.experimental import pallas as pl
from jax.experimental.pallas import tpu as pltpu

N = 2560
D = 256
ROW_BLK = 256
NUM_BLK = N // ROW_BLK
GROUPS = N // 5
TOPK = 110
LANES = 128
SUBL = N // LANES


def _body(in_ref, ptl_ref, gmask_ref, w_in_ref, b_in_ref, w_q_ref, b_q_ref,
          w_k_ref, b_k_ref, out_ref, q_s, k_s, p_s, dist_s):
    t = pl.program_id(0)

    @pl.when(t < 4)
    def flash_phase():
        pts = in_ref[t]
        x = jnp.dot(pts, w_in_ref[...], preferred_element_type=jnp.float32)
        x = x + b_in_ref[...]
        q = jnp.dot(x, w_q_ref[...], preferred_element_type=jnp.float32) + b_q_ref[...]
        k = jnp.dot(x, w_k_ref[...], preferred_element_type=jnp.float32) + b_k_ref[...]
        q_s[...] = q * jnp.float32(1.4426950408889634 / 16.0)
        k_s[...] = k

        ident = (lax.broadcasted_iota(jnp.int32, (ROW_BLK, ROW_BLK), 0) ==
                 lax.broadcasted_iota(jnp.int32, (ROW_BLK, ROW_BLK), 1))

        for i in range(NUM_BLK):
            qb = q_s[i * ROW_BLK:(i + 1) * ROW_BLK, :]
            s = lax.dot_general(qb, k_s[...], (((1,), (1,)), ((), ())),
                                preferred_element_type=jnp.float32)
            e = jnp.exp2(s)
            l = jnp.sum(e, axis=1, keepdims=True)
            sblk = s[:, i * ROW_BLK:(i + 1) * ROW_BLK]
            diag_s = jnp.sum(jnp.where(ident, sblk, 0.0),
                             axis=1, keepdims=True)
            p_s[i * ROW_BLK:(i + 1) * ROW_BLK, :] = jnp.exp2(diag_s) / l

        gs = jnp.dot(gmask_ref[...], p_s[...],
                     preferred_element_type=jnp.float32,
                     precision=lax.Precision.HIGHEST)
        gbest = jnp.max(gs)
        gidx = lax.broadcasted_iota(jnp.int32, (GROUPS, 1), 0)
        g = jnp.min(jnp.where(gs == gbest, gidx, GROUPS))

        group_pts = in_ref[t, pl.ds(g * 5, 5), :]
        cx = jnp.sum(group_pts[:, 0:1]) / 5.0
        cy = jnp.sum(group_pts[:, 1:2]) / 5.0
        cz = jnp.sum(group_pts[:, 2:3]) / 5.0

        dx = ptl_ref[t, 0] - cx
        dy = ptl_ref[t, 1] - cy
        dz = ptl_ref[t, 2] - cz
        dist_s[t] = jnp.sqrt(dx * dx + dy * dy + dz * dz)

    @pl.when(t == 4)
    def select_phase():
        lin = (lax.broadcasted_iota(jnp.int32, (SUBL, LANES), 0) * LANES +
               lax.broadcasted_iota(jnp.int32, (SUBL, LANES), 1))

        def pick(ti, carry):
            ds_ = list(carry)
            for b in range(4):
                db = ds_[b]
                mn = jnp.min(db)
                ib = jnp.min(jnp.where(db == mn, lin, N))
                out_ref[b, pl.ds(ti, 1), :] = in_ref[b, pl.ds(ib, 1), :]
                ds_[b] = jnp.where(lin == ib, jnp.inf, db)
            return tuple(ds_)

        lax.fori_loop(0, TOPK, pick,
                      tuple(dist_s[b] for b in range(4)))


@functools.partial(jax.jit, static_argnames=("interpret",))
def _run(in_mat, W_in, b_in, W_q, b_q, W_k, b_k, interpret=False):
    B = in_mat.shape[0]
    pts_lanes = in_mat.transpose(0, 2, 1).reshape(B, 3, SUBL, LANES)
    gcol = jnp.arange(N, dtype=jnp.int32)[None, :]
    grow = jnp.arange(GROUPS, dtype=jnp.int32)[:, None]
    gmask = jnp.where((gcol >= grow * 5) & (gcol < grow * 5 + 5), 1.0, 0.0)
    in_specs = [
        pl.BlockSpec((B, N, 3), lambda t: (0, 0, 0)),
        pl.BlockSpec((B, 3, SUBL, LANES), lambda t: (0, 0, 0, 0)),
        pl.BlockSpec((GROUPS, N), lambda t: (0, 0)),
        pl.BlockSpec((3, D), lambda t: (0, 0)),
        pl.BlockSpec((1, D), lambda t: (0, 0)),
        pl.BlockSpec((D, D), lambda t: (0, 0)),
        pl.BlockSpec((1, D), lambda t: (0, 0)),
        pl.BlockSpec((D, D), lambda t: (0, 0)),
        pl.BlockSpec((1, D), lambda t: (0, 0)),
    ]
    return pl.pallas_call(
        _body,
        grid=(5,),
        in_specs=in_specs,
        out_specs=pl.BlockSpec((B, TOPK, 3), lambda t: (0, 0, 0)),
        out_shape=jax.ShapeDtypeStruct((B, TOPK, 3), jnp.float32),
        scratch_shapes=[
            pltpu.VMEM((N, D), jnp.float32),
            pltpu.VMEM((N, D), jnp.float32),
            pltpu.VMEM((N, 1), jnp.float32),
            pltpu.VMEM((B, SUBL, LANES), jnp.float32),
        ],
        interpret=interpret,
    )(in_mat, pts_lanes, gmask, W_in, b_in.reshape(1, D), W_q,
      b_q.reshape(1, D), W_k, b_k.reshape(1, D))


def kernel(in_mat, W_in, b_in, W_q, b_q, W_k, b_k, W_v, b_v, W_o, b_o):
    del W_v, b_v, W_o, b_o
    return _run(in_mat, W_in, b_in, W_q, b_q, W_k, b_k)


# speedup vs baseline: 1.9691x; 1.8364x over previous
import functools

import jax
import jax.numpy as jnp
from jax import lax
from jax.experimental import pallas as pl
from jax.experimental.pallas import tpu as pltpu

N = 2560
D = 256
ROW_BLK = 256
NUM_BLK = N // ROW_BLK
GROUPS = N // 5
TOPK = 110
SEL_PAD = 112


def _body(in_ref, ptr_ref, gmask_ref, w_in_ref, b_in_ref, w_q_ref, b_q_ref,
          w_k_ref, b_k_ref, out_ref, q_s, k_s, p_s, rank_s):
    pts = in_ref[0]
    x = jnp.dot(pts, w_in_ref[...], preferred_element_type=jnp.float32)
    x = x + b_in_ref[...]
    q = jnp.dot(x, w_q_ref[...], preferred_element_type=jnp.float32) + b_q_ref[...]
    k = jnp.dot(x, w_k_ref[...], preferred_element_type=jnp.float32) + b_k_ref[...]
    q_s[...] = q * jnp.float32(1.4426950408889634 / 16.0)
    k_s[...] = k

    ident = (lax.broadcasted_iota(jnp.int32, (ROW_BLK, ROW_BLK), 0) ==
             lax.broadcasted_iota(jnp.int32, (ROW_BLK, ROW_BLK), 1))

    for i in range(NUM_BLK):
        qb = q_s[i * ROW_BLK:(i + 1) * ROW_BLK, :]
        s = lax.dot_general(qb, k_s[...], (((1,), (1,)), ((), ())),
                            preferred_element_type=jnp.float32)
        e = jnp.exp2(s)
        l = jnp.sum(e, axis=1, keepdims=True)
        sblk = s[:, i * ROW_BLK:(i + 1) * ROW_BLK]
        diag_s = jnp.sum(jnp.where(ident, sblk, 0.0),
                         axis=1, keepdims=True)
        p_s[i * ROW_BLK:(i + 1) * ROW_BLK, :] = jnp.exp2(diag_s) / l

    gs = jnp.dot(gmask_ref[...], p_s[...],
                 preferred_element_type=jnp.float32,
                 precision=lax.Precision.HIGHEST)
    gbest = jnp.max(gs)
    gidx = lax.broadcasted_iota(jnp.int32, (GROUPS, 1), 0)
    g = jnp.min(jnp.where(gs == gbest, gidx, GROUPS))

    group_pts = in_ref[0, pl.ds(g * 5, 5), :]
    cx = jnp.sum(group_pts[:, 0:1]) / 5.0
    cy = jnp.sum(group_pts[:, 1:2]) / 5.0
    cz = jnp.sum(group_pts[:, 2:3]) / 5.0

    dxc = pts[:, 0:1] - cx
    dyc = pts[:, 1:2] - cy
    dzc = pts[:, 2:3] - cz
    d_col = jnp.sqrt(dxc * dxc + dyc * dyc + dzc * dzc)
    dxr = ptr_ref[0, 0:1, :] - cx
    dyr = ptr_ref[0, 1:2, :] - cy
    dzr = ptr_ref[0, 2:3, :] - cz
    d_row = jnp.sqrt(dxr * dxr + dyr * dyr + dzr * dzr)

    ciota = lax.broadcasted_iota(jnp.int32, (ROW_BLK, N), 1)
    riota0 = lax.broadcasted_iota(jnp.int32, (ROW_BLK, N), 0)
    for rb in range(NUM_BLK):
        dcb = d_col[rb * ROW_BLK:(rb + 1) * ROW_BLK, :]
        lt = d_row < dcb
        eq = d_row == dcb
        ilt = ciota < (riota0 + rb * ROW_BLK)
        cf = jnp.where(lt | (eq & ilt), 1.0, 0.0)
        rank_s[rb * ROW_BLK:(rb + 1) * ROW_BLK, :] = (
            jnp.sum(cf, axis=1, keepdims=True))

    tio = lax.broadcasted_iota(jnp.int32, (1, SEL_PAD), 1).astype(jnp.float32)
    ot = jnp.where(rank_s[...] == tio, 1.0, 0.0)
    sel = lax.dot_general(ot, pts, (((0,), (0,)), ((), ())),
                          preferred_element_type=jnp.float32,
                          precision=lax.Precision.HIGHEST)
    out_ref[0] = sel[0:TOPK, :]


@functools.partial(jax.jit, static_argnames=("interpret",))
def _run(in_mat, W_in, b_in, W_q, b_q, W_k, b_k, interpret=False):
    B = in_mat.shape[0]
    pts_t = in_mat.transpose(0, 2, 1)
    gcol = jnp.arange(N, dtype=jnp.int32)[None, :]
    grow = jnp.arange(GROUPS, dtype=jnp.int32)[:, None]
    gmask = jnp.where((gcol >= grow * 5) & (gcol < grow * 5 + 5), 1.0, 0.0)
    in_specs = [
        pl.BlockSpec((1, N, 3), lambda b: (b, 0, 0)),
        pl.BlockSpec((1, 3, N), lambda b: (b, 0, 0)),
        pl.BlockSpec((GROUPS, N), lambda b: (0, 0)),
        pl.BlockSpec((3, D), lambda b: (0, 0)),
        pl.BlockSpec((1, D), lambda b: (0, 0)),
        pl.BlockSpec((D, D), lambda b: (0, 0)),
        pl.BlockSpec((1, D), lambda b: (0, 0)),
        pl.BlockSpec((D, D), lambda b: (0, 0)),
        pl.BlockSpec((1, D), lambda b: (0, 0)),
    ]
    return pl.pallas_call(
        _body,
        grid=(B,),
        in_specs=in_specs,
        out_specs=pl.BlockSpec((1, TOPK, 3), lambda b: (b, 0, 0)),
        out_shape=jax.ShapeDtypeStruct((B, TOPK, 3), jnp.float32),
        scratch_shapes=[
            pltpu.VMEM((N, D), jnp.float32),
            pltpu.VMEM((N, D), jnp.float32),
            pltpu.VMEM((N, 1), jnp.float32),
            pltpu.VMEM((N, 1), jnp.float32),
        ],
        interpret=interpret,
    )(in_mat, pts_t, gmask, W_in, b_in.reshape(1, D), W_q,
      b_q.reshape(1, D), W_k, b_k.reshape(1, D))


def kernel(in_mat, W_in, b_in, W_q, b_q, W_k, b_k, W_v, b_v, W_o, b_o):
    del W_v, b_v, W_o, b_o
    return _run(in_mat, W_in, b_in, W_q, b_q, W_k, b_k)


# speedup vs baseline: 1.9715x; 1.0012x over previous
import functools

import jax
import jax.numpy as jnp
from jax import lax
from jax.experimental import pallas as pl
from jax.experimental.pallas import tpu as pltpu

N = 2560
D = 256
ROW_BLK = 256
NUM_BLK = N // ROW_BLK
GROUPS = N // 5
TOPK = 110
SEL_PAD = 112


def _body(in_ref, ptr_ref, gmask_ref, w_in_ref, b_in_ref, w_q_ref, b_q_ref,
          w_k_ref, b_k_ref, out_ref, q_s, k_s, p_s, rank_s):
    pts = in_ref[0]
    x = jnp.dot(pts, w_in_ref[...], preferred_element_type=jnp.float32)
    x = x + b_in_ref[...]
    q = jnp.dot(x, w_q_ref[...], preferred_element_type=jnp.float32) + b_q_ref[...]
    k = jnp.dot(x, w_k_ref[...], preferred_element_type=jnp.float32) + b_k_ref[...]
    q_s[...] = q * jnp.float32(1.4426950408889634 / 16.0)
    k_s[...] = k

    ident = (lax.broadcasted_iota(jnp.int32, (ROW_BLK, ROW_BLK), 0) ==
             lax.broadcasted_iota(jnp.int32, (ROW_BLK, ROW_BLK), 1))

    for i in range(NUM_BLK):
        qb = q_s[i * ROW_BLK:(i + 1) * ROW_BLK, :]
        s = lax.dot_general(qb, k_s[...], (((1,), (1,)), ((), ())),
                            preferred_element_type=jnp.float32)
        e = jnp.exp2(s)
        l = jnp.sum(e, axis=1, keepdims=True)
        sblk = s[:, i * ROW_BLK:(i + 1) * ROW_BLK]
        diag_s = jnp.sum(jnp.where(ident, sblk, 0.0),
                         axis=1, keepdims=True)
        p_s[i * ROW_BLK:(i + 1) * ROW_BLK, :] = jnp.exp2(diag_s) / l

    gs = jnp.dot(gmask_ref[...], p_s[...],
                 preferred_element_type=jnp.float32,
                 precision=lax.Precision.HIGHEST)
    gbest = jnp.max(gs)
    gidx = lax.broadcasted_iota(jnp.int32, (GROUPS, 1), 0)
    g = jnp.min(jnp.where(gs == gbest, gidx, GROUPS))

    group_pts = in_ref[0, pl.ds(g * 5, 5), :]
    cx = jnp.sum(group_pts[:, 0:1]) / 5.0
    cy = jnp.sum(group_pts[:, 1:2]) / 5.0
    cz = jnp.sum(group_pts[:, 2:3]) / 5.0

    dxc = pts[:, 0:1] - cx
    dyc = pts[:, 1:2] - cy
    dzc = pts[:, 2:3] - cz
    d_col = jnp.sqrt(dxc * dxc + dyc * dyc + dzc * dzc)
    dxr = ptr_ref[0, 0:1, :] - cx
    dyr = ptr_ref[0, 1:2, :] - cy
    dzr = ptr_ref[0, 2:3, :] - cz
    d_row = jnp.sqrt(dxr * dxr + dyr * dyr + dzr * dzr)

    cd = (lax.broadcasted_iota(jnp.int32, (ROW_BLK, N), 1) -
          lax.broadcasted_iota(jnp.int32, (ROW_BLK, N), 0))
    for rb in range(NUM_BLK):
        dcb = d_col[rb * ROW_BLK:(rb + 1) * ROW_BLK, :]
        lt = d_row < dcb
        eq = d_row == dcb
        ilt = cd < rb * ROW_BLK
        cf = jnp.where(lt | (eq & ilt), 1.0, 0.0)
        rank_s[rb * ROW_BLK:(rb + 1) * ROW_BLK, :] = (
            jnp.sum(cf, axis=1, keepdims=True))

    tio = lax.broadcasted_iota(jnp.int32, (1, SEL_PAD), 1).astype(jnp.float32)
    ot = jnp.where(rank_s[...] == tio, 1.0, 0.0)
    sel = lax.dot_general(ot, pts, (((0,), (0,)), ((), ())),
                          preferred_element_type=jnp.float32,
                          precision=lax.Precision.HIGHEST)
    out_ref[0] = sel[0:TOPK, :]


@functools.partial(jax.jit, static_argnames=("interpret",))
def _run(in_mat, W_in, b_in, W_q, b_q, W_k, b_k, interpret=False):
    B = in_mat.shape[0]
    pts_t = in_mat.transpose(0, 2, 1)
    gcol = jnp.arange(N, dtype=jnp.int32)[None, :]
    grow = jnp.arange(GROUPS, dtype=jnp.int32)[:, None]
    gmask = jnp.where((gcol >= grow * 5) & (gcol < grow * 5 + 5), 1.0, 0.0)
    in_specs = [
        pl.BlockSpec((1, N, 3), lambda b: (b, 0, 0)),
        pl.BlockSpec((1, 3, N), lambda b: (b, 0, 0)),
        pl.BlockSpec((GROUPS, N), lambda b: (0, 0)),
        pl.BlockSpec((3, D), lambda b: (0, 0)),
        pl.BlockSpec((1, D), lambda b: (0, 0)),
        pl.BlockSpec((D, D), lambda b: (0, 0)),
        pl.BlockSpec((1, D), lambda b: (0, 0)),
        pl.BlockSpec((D, D), lambda b: (0, 0)),
        pl.BlockSpec((1, D), lambda b: (0, 0)),
    ]
    return pl.pallas_call(
        _body,
        grid=(B,),
        in_specs=in_specs,
        out_specs=pl.BlockSpec((1, TOPK, 3), lambda b: (b, 0, 0)),
        out_shape=jax.ShapeDtypeStruct((B, TOPK, 3), jnp.float32),
        scratch_shapes=[
            pltpu.VMEM((N, D), jnp.float32),
            pltpu.VMEM((N, D), jnp.float32),
            pltpu.VMEM((N, 1), jnp.float32),
            pltpu.VMEM((N, 1), jnp.float32),
        ],
        interpret=interpret,
    )(in_mat, pts_t, gmask, W_in, b_in.reshape(1, D), W_q,
      b_q.reshape(1, D), W_k, b_k.reshape(1, D))


def kernel(in_mat, W_in, b_in, W_q, b_q, W_k, b_k, W_v, b_v, W_o, b_o):
    del W_v, b_v, W_o, b_o
    return _run(in_mat, W_in, b_in, W_q, b_q, W_k, b_k)
